# baseline (speedup 1.0000x reference)
import functools

import jax
import jax.numpy as jnp
from jax import lax
from jax.experimental import pallas as pl
from jax.experimental.pallas import tpu as pltpu
from jax.experimental.pallas import tpu_sc as plsc

N = 10000
E = 320000
D = 128
LEAKY_SLOPE = 0.01

NC = 2
NS = 16
NW = NC * NS
EP = E // NW
CHUNK = 80
NCHUNK = EP // CHUNK
NROWS = 10240
TRASH = N
RPT = NROWS // NS

_mesh = plsc.VectorSubcoreMesh(core_axis_name="c", subcore_axis_name="s")


@functools.partial(
    pl.kernel,
    out_type=(
        jax.ShapeDtypeStruct((NC * NROWS,), jnp.float32),
        jax.ShapeDtypeStruct((NC * NROWS,), jnp.float32),
        jax.ShapeDtypeStruct((NC * NROWS,), jnp.float32),
    ),
    mesh=_mesh,
    scratch_types=[
        pltpu.VMEM((EP,), jnp.int32),
        pltpu.VMEM((EP,), jnp.int32),
        [pltpu.VMEM((CHUNK,), jnp.int32) for _ in range(2)],
        [pltpu.VMEM((CHUNK,), jnp.int32) for _ in range(2)],
        [pltpu.VMEM((CHUNK,), jnp.int32) for _ in range(2)],
        pltpu.VMEM((CHUNK,), jnp.float32),
        pltpu.VMEM((RPT,), jnp.float32),
        pltpu.VMEM_SHARED((NROWS,), jnp.float32),
        pltpu.VMEM_SHARED((NROWS,), jnp.float32),
        pltpu.VMEM_SHARED((NROWS,), jnp.float32),
        [pltpu.SemaphoreType.DMA for _ in range(2)],
        [pltpu.SemaphoreType.DMA for _ in range(2)],
        [pltpu.SemaphoreType.DMA for _ in range(2)],
    ],
)
def _deg_kernel(src_hbm, dst_hbm,
                rout_hbm, rin_hbm, selfc_hbm,
                srcv, dstv, rsidx, rdidx, sfidx, onesv, zv,
                sh_out, sh_in, sh_sf, osem, isem, fsem):
    cid = lax.axis_index("c")
    sid = lax.axis_index("s")
    wid = sid * NC + cid
    base = wid * EP

    pltpu.sync_copy(src_hbm.at[pl.ds(base, EP)], srcv)
    pltpu.sync_copy(dst_hbm.at[pl.ds(base, EP)], dstv)

    for g in range(RPT // 16):
        zv[pl.ds(g * 16, 16)] = jnp.zeros((16,), jnp.float32)
    pltpu.sync_copy(zv, sh_out.at[pl.ds(sid * RPT, RPT)])
    pltpu.sync_copy(zv, sh_in.at[pl.ds(sid * RPT, RPT)])
    pltpu.sync_copy(zv, sh_sf.at[pl.ds(sid * RPT, RPT)])
    for g in range(CHUNK // 16):
        onesv[pl.ds(g * 16, 16)] = jnp.ones((16,), jnp.float32)
    plsc.subcore_barrier()

    def compute_idx(j, p):
        off = j * CHUNK
        for g in range(CHUNK // 16):
            s16 = srcv[pl.ds(off + g * 16, 16)]
            d16 = dstv[pl.ds(off + g * 16, 16)]
            rsidx[p][pl.ds(g * 16, 16)] = s16
            rdidx[p][pl.ds(g * 16, 16)] = d16
            sfidx[p][pl.ds(g * 16, 16)] = jnp.where(s16 != d16, TRASH, s16)

    def fire(j, p):
        compute_idx(j, p)
        pltpu.async_copy(onesv, sh_out.at[rsidx[p]], osem[p], add=True)
        pltpu.async_copy(onesv, sh_in.at[rdidx[p]], isem[p], add=True)
        pltpu.async_copy(onesv, sh_sf.at[sfidx[p]], fsem[p], add=True)

    def wait(p):
        pltpu.make_async_copy(onesv, sh_out.at[rsidx[p]], osem[p]).wait()
        pltpu.make_async_copy(onesv, sh_in.at[rdidx[p]], isem[p]).wait()
        pltpu.make_async_copy(onesv, sh_sf.at[sfidx[p]], fsem[p]).wait()

    fire(0, 0)
    fire(1, 1)

    def body(t, carry):
        jb = 2 + t * 2
        wait(0)
        fire(jb, 0)
        wait(1)
        fire(jb + 1, 1)
        return carry

    lax.fori_loop(0, (NCHUNK - 2) // 2, body, 0)
    wait(0)
    fire(NCHUNK - 1, 0)
    wait(1)
    wait(0)
    plsc.subcore_barrier()

    out_off = cid * NROWS + sid * RPT
    pltpu.sync_copy(sh_out.at[pl.ds(sid * RPT, RPT)],
                    rout_hbm.at[pl.ds(out_off, RPT)])
    pltpu.sync_copy(sh_in.at[pl.ds(sid * RPT, RPT)],
                    rin_hbm.at[pl.ds(out_off, RPT)])
    pltpu.sync_copy(sh_sf.at[pl.ds(sid * RPT, RPT)],
                    selfc_hbm.at[pl.ds(out_off, RPT)])


@functools.partial(
    pl.kernel,
    out_type=jax.ShapeDtypeStruct((NC * NROWS, D), jnp.float32),
    mesh=_mesh,
    scratch_types=[
        [pltpu.VMEM((CHUNK,), jnp.int32) for _ in range(6)],
        [pltpu.VMEM((CHUNK,), jnp.int32) for _ in range(6)],
        [pltpu.VMEM((CHUNK, D), jnp.float32) for _ in range(4)],
        pltpu.VMEM_SHARED((NROWS, D), jnp.float32),
        [pltpu.SemaphoreType.DMA for _ in range(6)],
        [pltpu.SemaphoreType.DMA for _ in range(4)],
        [pltpu.SemaphoreType.DMA for _ in range(4)],
    ],
)
def _agg_kernel(h_hbm, src_hbm, dst_hbm, zeros_hbm, agg_hbm,
                srcc, dstc, rows, sh_agg, esem, gsem, ssem):
    cid = lax.axis_index("c")
    sid = lax.axis_index("s")
    wid = sid * NC + cid
    base = wid * EP

    pltpu.sync_copy(zeros_hbm.at[pl.ds(sid * RPT, RPT)],
                    sh_agg.at[pl.ds(sid * RPT, RPT)])

    def fire_eload(j, ph):
        m = ph % 6
        off = base + j * CHUNK
        pltpu.async_copy(src_hbm.at[pl.ds(off, CHUNK)], srcc[m], esem[m])
        pltpu.async_copy(dst_hbm.at[pl.ds(off, CHUNK)], dstc[m], esem[m])

    def wait_eload(j, ph):
        m = ph % 6
        off = base + j * CHUNK
        pltpu.make_async_copy(
            src_hbm.at[pl.ds(off, CHUNK)], srcc[m], esem[m]).wait()
        pltpu.make_async_copy(
            dst_hbm.at[pl.ds(off, CHUNK)], dstc[m], esem[m]).wait()

    def fire_gather(j, ph):
        m, r = ph % 6, ph % 4
        pltpu.async_copy(h_hbm.at[srcc[m]], rows[r], gsem[r])

    def wait_gather(j, ph):
        m, r = ph % 6, ph % 4
        pltpu.make_async_copy(h_hbm.at[srcc[m]], rows[r], gsem[r]).wait()

    def fire_scatter(j, ph):
        m, r = ph % 6, ph % 4
        pltpu.async_copy(rows[r], sh_agg.at[dstc[m]], ssem[r], add=True)

    def wait_scatter(j, ph):
        m, r = ph % 6, ph % 4
        pltpu.make_async_copy(rows[r], sh_agg.at[dstc[m]], ssem[r]).wait()

    plsc.subcore_barrier()

    for j in range(4):
        fire_eload(j, j)
    wait_eload(0, 0)
    fire_gather(0, 0)
    wait_eload(1, 1)
    fire_gather(1, 1)
    fire_eload(4, 4)
    wait_eload(2, 2)
    fire_gather(2, 2)
    wait_gather(0, 0)
    fire_scatter(0, 0)
    fire_eload(5, 5)
    wait_eload(3, 3)
    fire_gather(3, 3)
    wait_gather(1, 1)
    fire_scatter(1, 1)

    def stage(j, ph):
        wait_scatter(j - 2, ph - 2)
        fire_eload(j + 4, ph + 4)
        wait_eload(j + 2, ph + 2)
        fire_gather(j + 2, ph + 2)
        wait_gather(j, ph)
        fire_scatter(j, ph)

    def body(t, carry):
        jb = 2 + t * 12
        for k in range(12):
            stage(jb + k, 2 + k)
        return carry

    lax.fori_loop(0, 9, body, 0)
    for j in range(110, 125):
        wait_scatter(j - 2, j - 2)
        if j + 4 < NCHUNK:
            fire_eload(j + 4, j + 4)
        if j + 2 < NCHUNK:
            wait_eload(j + 2, j + 2)
            fire_gather(j + 2, j + 2)
        wait_gather(j, j)
        fire_scatter(j, j)
    wait_scatter(NCHUNK - 2, NCHUNK - 2)
    wait_scatter(NCHUNK - 1, NCHUNK - 1)
    plsc.subcore_barrier()

    out_off = cid * NROWS + sid * RPT
    pltpu.sync_copy(sh_agg.at[pl.ds(sid * RPT, RPT)],
                    agg_hbm.at[pl.ds(out_off, RPT)])


_BM = 1000
_BMF = 2000


def _mm_scale_body(x_ref, w_ref, rout_ref, sf_ref, h_ref):
    xw = jnp.dot(x_ref[...], w_ref[...], preferred_element_type=jnp.float32)
    deg = (rout_ref[0, :, 0] + rout_ref[1, :, 0]
           - sf_ref[0, :, 0] - sf_ref[1, :, 0] + 1.0)
    h_ref[...] = xw * lax.rsqrt(deg)[:, None]


def _mm_scale(x, W, rout, sf):
    return pl.pallas_call(
        _mm_scale_body,
        grid=(N // _BM,),
        in_specs=[
            pl.BlockSpec((_BM, D), lambda i: (i, 0)),
            pl.BlockSpec((D, D), lambda i: (0, 0)),
            pl.BlockSpec((NC, _BM, 1), lambda i: (0, i, 0)),
            pl.BlockSpec((NC, _BM, 1), lambda i: (0, i, 0)),
        ],
        out_specs=pl.BlockSpec((_BM, D), lambda i: (i, 0)),
        out_shape=jax.ShapeDtypeStruct((N, D), jnp.float32),
    )(x, W, rout, sf)


def _final_body(agg_ref, h_ref, rin_ref, sf_ref, b_ref, o_ref):
    sf = sf_ref[0, :, 0] + sf_ref[1, :, 0]
    deg = rin_ref[0, :, 0] + rin_ref[1, :, 0] - sf + 1.0
    s = agg_ref[0] + agg_ref[1] + (1.0 - sf)[:, None] * h_ref[...]
    out = s * lax.rsqrt(deg)[:, None] + b_ref[0, :]
    o_ref[...] = jnp.where(out >= 0, out, LEAKY_SLOPE * out)


def _final(agg, h, rin, sf, b):
    return pl.pallas_call(
        _final_body,
        grid=(N // _BMF,),
        in_specs=[
            pl.BlockSpec((NC, _BMF, D), lambda i: (0, i, 0)),
            pl.BlockSpec((_BMF, D), lambda i: (i, 0)),
            pl.BlockSpec((NC, _BMF, 1), lambda i: (0, i, 0)),
            pl.BlockSpec((NC, _BMF, 1), lambda i: (0, i, 0)),
            pl.BlockSpec((1, D), lambda i: (0, 0)),
        ],
        out_specs=pl.BlockSpec((_BMF, D), lambda i: (i, 0)),
        out_shape=jax.ShapeDtypeStruct((N, D), jnp.float32),
    )(agg, h, rin, sf, b)


def kernel(x, edge_index, W, b):
    src = edge_index[0]
    dst = edge_index[1]
    zeros = jnp.zeros((NROWS, D), jnp.float32)
    rout, rin, selfc = _deg_kernel(src, dst)
    rout = rout.reshape(NC, NROWS, 1)
    rin = rin.reshape(NC, NROWS, 1)
    selfc = selfc.reshape(NC, NROWS, 1)

    h = _mm_scale(x, W, rout, selfc)
    agg = _agg_kernel(h, src, dst, zeros)
    agg = agg.reshape(NC, NROWS, D)
    out = _final(agg, h, rin, selfc, b.reshape(1, D))
    return out


# speedup vs baseline: 1.8545x; 1.8545x over previous
import functools

import jax
import jax.numpy as jnp
from jax import lax
from jax.experimental import pallas as pl
from jax.experimental.pallas import tpu as pltpu
from jax.experimental.pallas import tpu_sc as plsc

N = 10000
E = 320000
D = 128
LEAKY_SLOPE = 0.01

NC = 2
NS = 16
NW = NC * NS
EP = E // NW
CHUNK = 80
NCHUNK = EP // CHUNK
NROWS = 10240
TRASH = N
RPT = NROWS // NS

_mesh = plsc.VectorSubcoreMesh(core_axis_name="c", subcore_axis_name="s")


@functools.partial(
    pl.kernel,
    out_type=(
        jax.ShapeDtypeStruct((NC * NROWS,), jnp.float32),
        jax.ShapeDtypeStruct((NC * NROWS,), jnp.float32),
    ),
    mesh=_mesh,
    scratch_types=[
        pltpu.VMEM((EP,), jnp.int32),
        pltpu.VMEM((EP,), jnp.int32),
        pltpu.VMEM((CHUNK,), jnp.int32),
        pltpu.VMEM((CHUNK,), jnp.int32),
        pltpu.VMEM((CHUNK,), jnp.float32),
        pltpu.VMEM((RPT,), jnp.float32),
        pltpu.VMEM_SHARED((NROWS,), jnp.float32),
        pltpu.VMEM_SHARED((NROWS,), jnp.float32),
    ],
)
def _deg_kernel(src_hbm, dst_hbm, dout_hbm, din_hbm,
                srcv, dstv, sidx, didx, onesv, zv, sh_out, sh_in):
    cid = lax.axis_index("c")
    sid = lax.axis_index("s")
    wid = sid * NC + cid
    base = wid * EP

    pltpu.sync_copy(src_hbm.at[pl.ds(base, EP)], srcv)
    pltpu.sync_copy(dst_hbm.at[pl.ds(base, EP)], dstv)

    for g in range(RPT // 16):
        zv[pl.ds(g * 16, 16)] = jnp.zeros((16,), jnp.float32)
    pltpu.sync_copy(zv, sh_out.at[pl.ds(sid * RPT, RPT)])
    pltpu.sync_copy(zv, sh_in.at[pl.ds(sid * RPT, RPT)])
    for g in range(CHUNK // 16):
        onesv[pl.ds(g * 16, 16)] = jnp.ones((16,), jnp.float32)
    plsc.subcore_barrier()

    def body(j, carry):
        off = j * CHUNK
        for g in range(CHUNK // 16):
            s16 = srcv[pl.ds(off + g * 16, 16)]
            d16 = dstv[pl.ds(off + g * 16, 16)]
            m = s16 != d16
            sidx[pl.ds(g * 16, 16)] = jnp.where(m, s16, TRASH)
            didx[pl.ds(g * 16, 16)] = jnp.where(m, d16, TRASH)
        pltpu.sync_copy(onesv, sh_out.at[sidx], add=True)
        pltpu.sync_copy(onesv, sh_in.at[didx], add=True)
        return carry

    lax.fori_loop(0, NCHUNK, body, 0)
    plsc.subcore_barrier()

    out_off = cid * NROWS + sid * RPT
    pltpu.sync_copy(sh_out.at[pl.ds(sid * RPT, RPT)],
                    dout_hbm.at[pl.ds(out_off, RPT)])
    pltpu.sync_copy(sh_in.at[pl.ds(sid * RPT, RPT)],
                    din_hbm.at[pl.ds(out_off, RPT)])


@functools.partial(
    pl.kernel,
    out_type=jax.ShapeDtypeStruct((NC * NROWS, D), jnp.float32),
    mesh=_mesh,
    scratch_types=[
        [pltpu.VMEM((CHUNK,), jnp.int32) for _ in range(6)],
        [pltpu.VMEM((CHUNK,), jnp.int32) for _ in range(6)],
        [pltpu.VMEM((CHUNK,), jnp.int32) for _ in range(4)],
        [pltpu.VMEM((CHUNK, D), jnp.float32) for _ in range(4)],
        pltpu.VMEM_SHARED((NROWS, D), jnp.float32),
        [pltpu.SemaphoreType.DMA for _ in range(6)],
        [pltpu.SemaphoreType.DMA for _ in range(4)],
        [pltpu.SemaphoreType.DMA for _ in range(4)],
    ],
)
def _agg_kernel(h_hbm, src_hbm, dst_hbm, zeros_hbm, agg_hbm,
                srcc, dstc, didx, rows, sh_agg, esem, gsem, ssem):
    cid = lax.axis_index("c")
    sid = lax.axis_index("s")
    wid = sid * NC + cid
    base = wid * EP

    pltpu.sync_copy(zeros_hbm.at[pl.ds(sid * RPT, RPT)],
                    sh_agg.at[pl.ds(sid * RPT, RPT)])

    def fire_eload(j, ph):
        m = ph % 6
        off = base + j * CHUNK
        pltpu.async_copy(src_hbm.at[pl.ds(off, CHUNK)], srcc[m], esem[m])
        pltpu.async_copy(dst_hbm.at[pl.ds(off, CHUNK)], dstc[m], esem[m])

    def wait_eload(j, ph):
        m = ph % 6
        off = base + j * CHUNK
        pltpu.make_async_copy(
            src_hbm.at[pl.ds(off, CHUNK)], srcc[m], esem[m]).wait()
        pltpu.make_async_copy(
            dst_hbm.at[pl.ds(off, CHUNK)], dstc[m], esem[m]).wait()

    def fire_gather(j, ph):
        m, r = ph % 6, ph % 4
        pltpu.async_copy(h_hbm.at[srcc[m]], rows[r], gsem[r])

    def wait_gather(j, ph):
        m, r = ph % 6, ph % 4
        pltpu.make_async_copy(h_hbm.at[srcc[m]], rows[r], gsem[r]).wait()

    def fire_scatter(j, ph):
        m, r = ph % 6, ph % 4
        for g in range(CHUNK // 16):
            s16 = srcc[m][pl.ds(g * 16, 16)]
            d16 = dstc[m][pl.ds(g * 16, 16)]
            didx[r][pl.ds(g * 16, 16)] = jnp.where(s16 != d16, d16, TRASH)
        pltpu.async_copy(rows[r], sh_agg.at[didx[r]], ssem[r], add=True)

    def wait_scatter(j, ph):
        m, r = ph % 6, ph % 4
        pltpu.make_async_copy(rows[r], sh_agg.at[didx[r]], ssem[r]).wait()

    plsc.subcore_barrier()

    for j in range(4):
        fire_eload(j, j)
    wait_eload(0, 0)
    fire_gather(0, 0)
    wait_eload(1, 1)
    fire_gather(1, 1)
    fire_eload(4, 4)
    wait_eload(2, 2)
    fire_gather(2, 2)
    wait_gather(0, 0)
    fire_scatter(0, 0)
    fire_eload(5, 5)
    wait_eload(3, 3)
    fire_gather(3, 3)
    wait_gather(1, 1)
    fire_scatter(1, 1)

    def stage(j, ph):
        wait_scatter(j - 2, ph - 2)
        fire_eload(j + 4, ph + 4)
        wait_eload(j + 2, ph + 2)
        fire_gather(j + 2, ph + 2)
        wait_gather(j, ph)
        fire_scatter(j, ph)

    def body(t, carry):
        jb = 2 + t * 12
        for k in range(12):
            stage(jb + k, 2 + k)
        return carry

    lax.fori_loop(0, 9, body, 0)
    for j in range(110, 125):
        wait_scatter(j - 2, j - 2)
        if j + 4 < NCHUNK:
            fire_eload(j + 4, j + 4)
        if j + 2 < NCHUNK:
            wait_eload(j + 2, j + 2)
            fire_gather(j + 2, j + 2)
        wait_gather(j, j)
        fire_scatter(j, j)
    wait_scatter(NCHUNK - 2, NCHUNK - 2)
    wait_scatter(NCHUNK - 1, NCHUNK - 1)
    plsc.subcore_barrier()

    out_off = cid * NROWS + sid * RPT
    pltpu.sync_copy(sh_agg.at[pl.ds(sid * RPT, RPT)],
                    agg_hbm.at[pl.ds(out_off, RPT)])


_BM = 1000
_BMF = 2000


def _mm_scale_body(x_ref, w_ref, dout_ref, h_ref):
    xw = jnp.dot(x_ref[...], w_ref[...], preferred_element_type=jnp.float32)
    deg = dout_ref[0, :, 0] + dout_ref[1, :, 0] + 1.0
    h_ref[...] = xw * lax.rsqrt(deg)[:, None]


def _mm_scale(x, W, dout):
    return pl.pallas_call(
        _mm_scale_body,
        grid=(N // _BM,),
        in_specs=[
            pl.BlockSpec((_BM, D), lambda i: (i, 0)),
            pl.BlockSpec((D, D), lambda i: (0, 0)),
            pl.BlockSpec((NC, _BM, 1), lambda i: (0, i, 0)),
        ],
        out_specs=pl.BlockSpec((_BM, D), lambda i: (i, 0)),
        out_shape=jax.ShapeDtypeStruct((N, D), jnp.float32),
    )(x, W, dout)


def _final_body(agg_ref, h_ref, din_ref, b_ref, o_ref):
    deg = din_ref[0, :, 0] + din_ref[1, :, 0] + 1.0
    s = agg_ref[0] + agg_ref[1] + h_ref[...]
    out = s * lax.rsqrt(deg)[:, None] + b_ref[0, :]
    o_ref[...] = jnp.where(out >= 0, out, LEAKY_SLOPE * out)


def _final(agg, h, din, b):
    return pl.pallas_call(
        _final_body,
        grid=(N // _BMF,),
        in_specs=[
            pl.BlockSpec((NC, _BMF, D), lambda i: (0, i, 0)),
            pl.BlockSpec((_BMF, D), lambda i: (i, 0)),
            pl.BlockSpec((NC, _BMF, 1), lambda i: (0, i, 0)),
            pl.BlockSpec((1, D), lambda i: (0, 0)),
        ],
        out_specs=pl.BlockSpec((_BMF, D), lambda i: (i, 0)),
        out_shape=jax.ShapeDtypeStruct((N, D), jnp.float32),
    )(agg, h, din, b)


def kernel(x, edge_index, W, b):
    src = edge_index[0]
    dst = edge_index[1]
    zeros = jnp.zeros((NROWS, D), jnp.float32)
    dout, din = _deg_kernel(src, dst)
    dout = dout.reshape(NC, NROWS, 1)
    din = din.reshape(NC, NROWS, 1)

    h = _mm_scale(x, W, dout)
    agg = _agg_kernel(h, src, dst, zeros)
    agg = agg.reshape(NC, NROWS, D)
    out = _final(agg, h, din, b.reshape(1, D))
    return out
